# baseline (speedup 1.0000x reference)
import functools

import jax
import jax.numpy as jnp
from jax import lax
from jax.experimental import pallas as pl
from jax.experimental.pallas import tpu as pltpu
from jax.experimental.pallas import tpu_sc as plsc

B = 16384
F = 26
V = 1000000

NC = 2
NS = 16
NW = NC * NS
BPW = B // NW
CH = 128
RPF = BPW // CH
NROW = F * RPF
LANES = 16
VR = V // LANES
NCHUNK = BPW // LANES


def _build_sc_call():
    mesh = plsc.VectorSubcoreMesh(core_axis_name="c", subcore_axis_name="s")

    @functools.partial(
        pl.kernel,
        mesh=mesh,
        compiler_params=pltpu.CompilerParams(
            needs_layout_passes=False, use_tc_tiling_on_sc=False
        ),
        out_type=jax.ShapeDtypeStruct((B,), jnp.float32),
        scratch_types=[
            pltpu.VMEM((NROW, CH), jnp.int32),
            pltpu.VMEM((RPF, CH), jnp.int32),
            pltpu.VMEM((RPF, CH), jnp.int32),
            pltpu.VMEM((BPW, LANES), jnp.float32),
            pltpu.VMEM((BPW, LANES), jnp.float32),
            pltpu.VMEM((BPW,), jnp.float32),
            pltpu.SemaphoreType.DMA,
            pltpu.SemaphoreType.DMA,
        ],
    )
    def sc_body(w2_hbm, x_hbm, out_hbm,
                x_v, row_a, row_b, buf_a, buf_b, acc_v, sem_a, sem_b):
        wid = lax.axis_index("s") * NC + lax.axis_index("c")

        pltpu.sync_copy(x_hbm.at[wid], x_v)

        def compute_rows(f, row_v):
            def rbody(j, carry):
                xv = x_v[f * RPF + j // 8, pl.ds((j % 8) * LANES, LANES)]
                row_v[j // 8, pl.ds((j % 8) * LANES, LANES)] = (
                    (xv >> 4) + f * VR
                )
                return carry
            lax.fori_loop(0, RPF * 8, rbody, 0)

        def fire(row_v, buf, sem):
            for r in range(RPF):
                pltpu.async_copy(
                    w2_hbm.at[row_v.at[r]], buf.at[pl.ds(r * CH, CH)], sem
                )

        def drain(buf, sem):
            pltpu.make_async_copy(w2_hbm.at[pl.ds(0, BPW)], buf, sem).wait()

        def extract(f, buf):
            def ebody(c, carry):
                xv = x_v[f * RPF + c // 8, pl.ds((c % 8) * LANES, LANES)]
                lanes = xv & 15
                rows16 = c * LANES + lax.iota(jnp.int32, LANES)
                vals = plsc.load_gather(buf, [rows16, lanes])
                prev = acc_v[pl.ds(c * LANES, LANES)]
                acc_v[pl.ds(c * LANES, LANES)] = prev + vals
                return carry
            lax.fori_loop(0, NCHUNK, ebody, 0)

        def zero_acc():
            def zbody(c, carry):
                acc_v[pl.ds(c * LANES, LANES)] = jnp.zeros(
                    (LANES,), jnp.float32
                )
                return carry
            lax.fori_loop(0, NCHUNK, zbody, 0)

        zero_acc()
        compute_rows(0, row_a)
        fire(row_a, buf_a, sem_a)
        for f in range(F):
            even = f % 2 == 0
            cur_row, cur_buf, cur_sem = (
                (row_a, buf_a, sem_a) if even else (row_b, buf_b, sem_b)
            )
            nxt_row, nxt_buf, nxt_sem = (
                (row_b, buf_b, sem_b) if even else (row_a, buf_a, sem_a)
            )
            if f + 1 < F:
                compute_rows(f + 1, nxt_row)
                fire(nxt_row, nxt_buf, nxt_sem)
            drain(cur_buf, cur_sem)
            extract(f, cur_buf)

        def sbody(c, carry):
            a = acc_v[pl.ds(c * LANES, LANES)]
            acc_v[pl.ds(c * LANES, LANES)] = 1.0 / (1.0 + jnp.exp(-a))
            return carry
        lax.fori_loop(0, NCHUNK, sbody, 0)

        pltpu.sync_copy(acc_v, out_hbm.at[pl.ds(wid * BPW, BPW)])

    return sc_body


_sc_call = _build_sc_call()


@jax.jit
def kernel(X, W_linear):
    x3 = X.T.reshape(F, NW, BPW).transpose(1, 0, 2).reshape(NW, NROW, CH)
    w2 = W_linear.reshape(F * VR, LANES)
    out = _sc_call(w2, x3)
    return out.reshape(B, 1)


# speedup vs baseline: 1.0027x; 1.0027x over previous
import functools

import jax
import jax.numpy as jnp
from jax import lax
from jax.experimental import pallas as pl
from jax.experimental.pallas import tpu as pltpu
from jax.experimental.pallas import tpu_sc as plsc

B = 16384
F = 26
V = 1000000

NC = 2
NS = 16
NW = NC * NS
BPW = B // NW
CH = 128
RPF = BPW // CH
NROW = F * RPF
LANES = 16
VR = V // LANES
NCHUNK = BPW // LANES


def _build_sc_call():
    mesh = plsc.VectorSubcoreMesh(core_axis_name="c", subcore_axis_name="s")

    @functools.partial(
        pl.kernel,
        mesh=mesh,
        compiler_params=pltpu.CompilerParams(
            needs_layout_passes=False,
            use_tc_tiling_on_sc=False,
            skip_device_barrier=True,
        ),
        out_type=jax.ShapeDtypeStruct((B,), jnp.float32),
        scratch_types=[
            pltpu.VMEM((NROW, CH), jnp.int32),
            pltpu.VMEM((RPF, CH), jnp.int32),
            pltpu.VMEM((RPF, CH), jnp.int32),
            pltpu.VMEM((BPW, LANES), jnp.float32),
            pltpu.VMEM((BPW, LANES), jnp.float32),
            pltpu.VMEM((BPW,), jnp.float32),
            pltpu.SemaphoreType.DMA,
            pltpu.SemaphoreType.DMA,
        ],
    )
    def sc_body(w2_hbm, x_hbm, out_hbm,
                x_v, row_a, row_b, buf_a, buf_b, acc_v, sem_a, sem_b):
        wid = lax.axis_index("s") * NC + lax.axis_index("c")

        pltpu.sync_copy(x_hbm.at[wid], x_v)

        def compute_rows(f, row_v):
            def rbody(j, carry):
                xv = x_v[f * RPF + j // 8, pl.ds((j % 8) * LANES, LANES)]
                row_v[j // 8, pl.ds((j % 8) * LANES, LANES)] = (
                    (xv >> 4) + f * VR
                )
                return carry
            lax.fori_loop(0, RPF * 8, rbody, 0)

        def fire(row_v, buf, sem):
            for r in range(RPF):
                pltpu.async_copy(
                    w2_hbm.at[row_v.at[r]], buf.at[pl.ds(r * CH, CH)], sem
                )

        def drain(buf, sem):
            pltpu.make_async_copy(w2_hbm.at[pl.ds(0, BPW)], buf, sem).wait()

        def extract(f, buf):
            def ebody(c, carry):
                xv = x_v[f * RPF + c // 8, pl.ds((c % 8) * LANES, LANES)]
                lanes = xv & 15
                rows16 = c * LANES + lax.iota(jnp.int32, LANES)
                vals = plsc.load_gather(buf, [rows16, lanes])
                prev = acc_v[pl.ds(c * LANES, LANES)]
                acc_v[pl.ds(c * LANES, LANES)] = prev + vals
                return carry
            lax.fori_loop(0, NCHUNK, ebody, 0)

        def zero_acc():
            def zbody(c, carry):
                acc_v[pl.ds(c * LANES, LANES)] = jnp.zeros(
                    (LANES,), jnp.float32
                )
                return carry
            lax.fori_loop(0, NCHUNK, zbody, 0)

        zero_acc()
        compute_rows(0, row_a)
        fire(row_a, buf_a, sem_a)
        for f in range(F):
            even = f % 2 == 0
            cur_row, cur_buf, cur_sem = (
                (row_a, buf_a, sem_a) if even else (row_b, buf_b, sem_b)
            )
            nxt_row, nxt_buf, nxt_sem = (
                (row_b, buf_b, sem_b) if even else (row_a, buf_a, sem_a)
            )
            if f + 1 < F:
                compute_rows(f + 1, nxt_row)
                fire(nxt_row, nxt_buf, nxt_sem)
            drain(cur_buf, cur_sem)
            extract(f, cur_buf)

        def sbody(c, carry):
            a = acc_v[pl.ds(c * LANES, LANES)]
            acc_v[pl.ds(c * LANES, LANES)] = 1.0 / (1.0 + jnp.exp(-a))
            return carry
        lax.fori_loop(0, NCHUNK, sbody, 0)

        pltpu.sync_copy(acc_v, out_hbm.at[pl.ds(wid * BPW, BPW)])

    return sc_body


_sc_call = _build_sc_call()


@jax.jit
def kernel(X, W_linear):
    x3 = X.T.reshape(F, NW, BPW).transpose(1, 0, 2).reshape(NW, NROW, CH)
    w2 = W_linear.reshape(F * VR, LANES)
    out = _sc_call(w2, x3)
    return out.reshape(B, 1)


# speedup vs baseline: 1.0042x; 1.0015x over previous
import functools

import jax
import jax.numpy as jnp
from jax import lax
from jax.experimental import pallas as pl
from jax.experimental.pallas import tpu as pltpu
from jax.experimental.pallas import tpu_sc as plsc

B = 16384
F = 26
V = 1000000

NC = 2
NS = 16
NW = NC * NS
BPW = B // NW
LANES = 16
VR = V // LANES
NP = 4
PH = F * BPW // NP
PV = PH // LANES
NCHUNK = BPW // LANES


def _build_sc_call():
    mesh = plsc.VectorSubcoreMesh(core_axis_name="c", subcore_axis_name="s")

    @functools.partial(
        pl.kernel,
        mesh=mesh,
        compiler_params=pltpu.CompilerParams(
            needs_layout_passes=False,
            use_tc_tiling_on_sc=False,
            skip_device_barrier=True,
        ),
        out_type=jax.ShapeDtypeStruct((B,), jnp.float32),
        scratch_types=[
            pltpu.VMEM((2, PH), jnp.int32),
            pltpu.VMEM((2, PH), jnp.int32),
            pltpu.VMEM((2, PH, LANES), jnp.float32),
            pltpu.VMEM((BPW,), jnp.float32),
            pltpu.SemaphoreType.DMA,
            pltpu.SemaphoreType.DMA,
        ],
    )
    def sc_body(w2_hbm, x_hbm, out_hbm, x_v, rows_v, buf, acc_v, sem_a, sem_b):
        wid = lax.axis_index("s") * NC + lax.axis_index("c")
        sems = (sem_a, sem_b)

        def prep_and_fire(p):
            s = p % 2
            pltpu.sync_copy(x_hbm.at[wid, pl.ds(p * PH, PH)], x_v.at[s])

            def rbody(i, carry):
                xv = x_v[s, pl.ds(i * LANES, LANES)]
                f = (i + p * PV) >> 5
                rows_v[s, pl.ds(i * LANES, LANES)] = (xv >> 4) + f * VR
                return carry

            lax.fori_loop(0, PV, rbody, 0)
            pltpu.async_copy(w2_hbm.at[rows_v.at[s]], buf.at[s], sems[s])

        def drain(p):
            s = p % 2
            pltpu.make_async_copy(
                w2_hbm.at[pl.ds(0, PH)], buf.at[s], sems[s]
            ).wait()

        def extract(p):
            s = p % 2

            def ebody(c, carry):
                xv = x_v[s, pl.ds(c * LANES, LANES)]
                lanes = xv & 15
                rows16 = c * LANES + lax.iota(jnp.int32, LANES)
                vals = plsc.load_gather(buf.at[s], [rows16, lanes])
                off = (p * PH + c * LANES) & (BPW - 1)
                prev = acc_v[pl.ds(off, LANES)]
                acc_v[pl.ds(off, LANES)] = prev + vals
                return carry

            lax.fori_loop(0, PV, ebody, 0)

        def zbody(c, carry):
            acc_v[pl.ds(c * LANES, LANES)] = jnp.zeros((LANES,), jnp.float32)
            return carry

        lax.fori_loop(0, NCHUNK, zbody, 0)

        prep_and_fire(0)
        for p in range(NP):
            if p + 1 < NP:
                prep_and_fire(p + 1)
            drain(p)
            extract(p)

        def sbody(c, carry):
            a = acc_v[pl.ds(c * LANES, LANES)]
            acc_v[pl.ds(c * LANES, LANES)] = 1.0 / (1.0 + jnp.exp(-a))
            return carry

        lax.fori_loop(0, NCHUNK, sbody, 0)

        pltpu.sync_copy(acc_v, out_hbm.at[pl.ds(wid * BPW, BPW)])

    return sc_body


_sc_call = _build_sc_call()


@jax.jit
def kernel(X, W_linear):
    x2 = X.T.reshape(F, NW, BPW).transpose(1, 0, 2).reshape(NW, F * BPW)
    w2 = W_linear.reshape(F * VR, LANES)
    out = _sc_call(w2, x2)
    return out.reshape(B, 1)


# speedup vs baseline: 1.0081x; 1.0039x over previous
import functools

import jax
import jax.numpy as jnp
from jax import lax
from jax.experimental import pallas as pl
from jax.experimental.pallas import tpu as pltpu
from jax.experimental.pallas import tpu_sc as plsc

B = 16384
F = 26
V = 1000000

NC = 2
NS = 16
NW = NC * NS
BPW = B // NW
LANES = 16
NCHUNK = BPW // LANES


def _build_sc_call():
    mesh = plsc.VectorSubcoreMesh(core_axis_name="c", subcore_axis_name="s")

    @functools.partial(
        pl.kernel,
        mesh=mesh,
        compiler_params=pltpu.CompilerParams(
            needs_layout_passes=False,
            use_tc_tiling_on_sc=False,
            skip_device_barrier=True,
        ),
        out_type=jax.ShapeDtypeStruct((B,), jnp.float32),
        scratch_types=[
            pltpu.VMEM((F, BPW), jnp.int32),
            pltpu.VMEM((F, BPW), jnp.float32),
            pltpu.VMEM((BPW,), jnp.float32),
            pltpu.SemaphoreType.DMA,
        ],
    )
    def sc_body(w_hbm, x_hbm, out_hbm, x_v, buf, acc_v, sem):
        wid = lax.axis_index("s") * NC + lax.axis_index("c")

        pltpu.sync_copy(x_hbm.at[wid], x_v)

        for f in range(F):
            pltpu.async_copy(w_hbm.at[f].at[x_v.at[f]], buf.at[f], sem)

        for f in range(F):
            pltpu.make_async_copy(
                w_hbm.at[f].at[x_v.at[f]], buf.at[f], sem
            ).wait()

        def rbody(c, carry):
            acc = buf[0, pl.ds(c * LANES, LANES)]
            for f in range(1, F):
                acc = acc + buf[f, pl.ds(c * LANES, LANES)]
            acc_v[pl.ds(c * LANES, LANES)] = 1.0 / (1.0 + jnp.exp(-acc))
            return carry

        lax.fori_loop(0, NCHUNK, rbody, 0)

        pltpu.sync_copy(acc_v, out_hbm.at[pl.ds(wid * BPW, BPW)])

    return sc_body


_sc_call = _build_sc_call()


@jax.jit
def kernel(X, W_linear):
    x3 = X.T.reshape(F, NW, BPW).transpose(1, 0, 2)
    out = _sc_call(W_linear, x3)
    return out.reshape(B, 1)


# speedup vs baseline: 3.5312x; 3.5028x over previous
import functools

import jax
import jax.numpy as jnp
from jax import lax
from jax.experimental import pallas as pl
from jax.experimental.pallas import tpu as pltpu
from jax.experimental.pallas import tpu_sc as plsc

B = 16384
F = 26
V = 1000000

NC = 2
NS = 16
NW = NC * NS
BPW = B // NW
LANES = 16
NCHUNK = BPW // LANES


def _build_sc_call():
    mesh = plsc.VectorSubcoreMesh(core_axis_name="c", subcore_axis_name="s")

    @functools.partial(
        pl.kernel,
        mesh=mesh,
        compiler_params=pltpu.CompilerParams(
            needs_layout_passes=False,
            use_tc_tiling_on_sc=False,
            skip_device_barrier=True,
        ),
        out_type=jax.ShapeDtypeStruct((B,), jnp.float32),
        scratch_types=[
            pltpu.VMEM((F, BPW), jnp.int32),
            pltpu.VMEM((F, BPW), jnp.float32),
            pltpu.VMEM((BPW,), jnp.float32),
            pltpu.SemaphoreType.DMA,
        ],
    )
    def sc_body(*refs):
        w_refs = refs[:F]
        x_hbm, out_hbm, x_v, buf, acc_v, sem = refs[F:]
        wid = lax.axis_index("s") * NC + lax.axis_index("c")

        pltpu.sync_copy(x_hbm.at[wid], x_v)

        for f in range(F):
            pltpu.async_copy(w_refs[f].at[x_v.at[f]], buf.at[f], sem)

        for f in range(F):
            pltpu.make_async_copy(
                w_refs[f].at[x_v.at[f]], buf.at[f], sem
            ).wait()

        def rbody(c, carry):
            acc = buf[0, pl.ds(c * LANES, LANES)]
            for f in range(1, F):
                acc = acc + buf[f, pl.ds(c * LANES, LANES)]
            acc_v[pl.ds(c * LANES, LANES)] = 1.0 / (1.0 + jnp.exp(-acc))
            return carry

        lax.fori_loop(0, NCHUNK, rbody, 0)

        pltpu.sync_copy(acc_v, out_hbm.at[pl.ds(wid * BPW, BPW)])

    return sc_body


_sc_call = _build_sc_call()


@jax.jit
def kernel(X, W_linear):
    x3 = X.T.reshape(F, NW, BPW).transpose(1, 0, 2)
    w_rows = [W_linear[f] for f in range(F)]
    out = _sc_call(*w_rows, x3)
    return out.reshape(B, 1)
